# Initial kernel scaffold; baseline (speedup 1.0000x reference)
import jax
import jax.numpy as jnp
from jax.experimental import pallas as pl


def kernel(x, edge_index, W, b):
    raise NotImplementedError("write your pallas kernel here")


# speedup vs baseline: 6.4399x; 6.4399x over previous
import jax
import jax.numpy as jnp
from jax import lax
from jax.experimental import pallas as pl
from jax.experimental.pallas import tpu as pltpu
from jax.experimental.pallas import tpu_sc as plsc

N_NODES = 10000
N_EDGES = 320000
EMB = 128

NC = 2
NS = 16
NW = NC * NS
EDGES_PER_TILE = N_EDGES // NW
CHUNK = 80
STEPS = EDGES_PER_TILE // CHUNK
N_PAD = 10240
ROWS_PER_TILE = N_PAD // NS
ZROWS = 64
HROWS = N_PAD // EMB
VW = 16


def _fill_rows(ref, nrows, ncols, value):
    v = jnp.full((VW,), value, jnp.float32)

    def body(r, carry):
        for j in range(ncols // VW):
            ref[r, pl.ds(j * VW, VW)] = v
        return carry

    lax.fori_loop(0, nrows, body, 0)


def _sc_body(x_hbm, src_hbm, dst_hbm, acc_out, deg_out,
             src_v, dst_v, rows_v, zrow_v, hist_v, iota_v, sem,
             acc_sp, deg_sp):
    c = lax.axis_index("c")
    s = lax.axis_index("s")
    wid = c * NS + s

    _fill_rows(zrow_v, ZROWS, EMB, 0.0)
    row0 = s * ROWS_PER_TILE
    for k in range(ROWS_PER_TILE // ZROWS):
        pltpu.sync_copy(zrow_v, acc_sp.at[pl.ds(row0 + k * ZROWS, ZROWS), :])
    _fill_rows(hist_v, HROWS, EMB, 0.0)
    @pl.when(s < HROWS // 8)
    def _zero_deg():
        pltpu.sync_copy(zrow_v.at[pl.ds(0, 8), :],
                        deg_sp.at[pl.ds(s * 8, 8), :])
    for j in range(HROWS // VW):
        iota_v[pl.ds(j * VW, VW)] = lax.iota(jnp.int32, VW) + j * VW
    plsc.subcore_barrier()

    ebase = wid * EDGES_PER_TILE

    def step(i, carry):
        off = ebase + i * CHUNK
        pltpu.sync_copy(src_hbm.at[pl.ds(off, CHUNK)], src_v)
        pltpu.sync_copy(dst_hbm.at[pl.ds(off, CHUNK)], dst_v)
        pltpu.async_copy(x_hbm.at[src_v], rows_v, sem).wait()
        pltpu.sync_copy(rows_v, acc_sp.at[dst_v], add=True)
        ones = jnp.full((VW,), 1.0, jnp.float32)
        for j in range(CHUNK // VW):
            d = dst_v[pl.ds(j * VW, VW)]
            plsc.addupdate_scatter(
                hist_v,
                (lax.shift_right_logical(d, 7), lax.bitwise_and(d, 127)),
                ones)
        return carry

    lax.fori_loop(0, STEPS, step, 0)
    pltpu.sync_copy(hist_v, deg_sp.at[iota_v], add=True)
    plsc.subcore_barrier()

    out0 = c * N_PAD + row0
    pltpu.sync_copy(acc_sp.at[pl.ds(row0, ROWS_PER_TILE), :],
                    acc_out.at[pl.ds(out0, ROWS_PER_TILE), :])
    @pl.when(s < HROWS // 8)
    def _copy_deg():
        pltpu.sync_copy(deg_sp.at[pl.ds(s * 8, 8), :],
                        deg_out.at[pl.ds(c * HROWS + s * 8, 8), :])


def _sc_aggregate(x, src, dst):
    mesh = plsc.VectorSubcoreMesh(core_axis_name="c", subcore_axis_name="s")
    f = pl.kernel(
        _sc_body,
        out_type=(
            jax.ShapeDtypeStruct((NC * N_PAD, EMB), jnp.float32),
            jax.ShapeDtypeStruct((NC * HROWS, EMB), jnp.float32),
        ),
        mesh=mesh,
        compiler_params=pltpu.CompilerParams(needs_layout_passes=False),
        scratch_types=[
            pltpu.VMEM((CHUNK,), jnp.int32),
            pltpu.VMEM((CHUNK,), jnp.int32),
            pltpu.VMEM((CHUNK, EMB), jnp.float32),
            pltpu.VMEM((ZROWS, EMB), jnp.float32),
            pltpu.VMEM((HROWS, EMB), jnp.float32),
            pltpu.VMEM((HROWS,), jnp.int32),
            pltpu.SemaphoreType.DMA,
            pltpu.VMEM_SHARED((N_PAD, EMB), jnp.float32),
            pltpu.VMEM_SHARED((HROWS, EMB), jnp.float32),
        ],
    )
    return f(x, src, dst)


def _tc_body(x_ref, acc_ref, deg_ref, w_ref, b_ref, out_ref):
    neigh = acc_ref[0] + acc_ref[1]
    deg = deg_ref[:, 0:1] + deg_ref[:, 1:2]
    agg = (x_ref[...] + neigh) * lax.rsqrt(deg + 1.0)
    h = lax.dot_general(agg, w_ref[...], (((1,), (1,)), ((), ())),
                        preferred_element_type=jnp.float32) + b_ref[...]
    out_ref[...] = jnp.where(h > 0, h, 0.2 * h)


def _tc_finish(x, acc, deg_t, W, b):
    R = 2000
    grid = (N_NODES // R,)
    return pl.pallas_call(
        _tc_body,
        grid=grid,
        in_specs=[
            pl.BlockSpec((R, EMB), lambda i: (i, 0)),
            pl.BlockSpec((NC, R, EMB), lambda i: (0, i, 0)),
            pl.BlockSpec((R, NC), lambda i: (i, 0)),
            pl.BlockSpec((EMB, EMB), lambda i: (0, 0)),
            pl.BlockSpec((1, EMB), lambda i: (0, 0)),
        ],
        out_specs=pl.BlockSpec((R, EMB), lambda i: (i, 0)),
        out_shape=jax.ShapeDtypeStruct((N_NODES, EMB), jnp.float32),
    )(x, acc, deg_t, W, b)


@jax.jit
def kernel(x, edge_index, W, b):
    src = edge_index[0]
    dst = edge_index[1]
    acc, deg = _sc_aggregate(x, src, dst)
    acc = acc.reshape(NC, N_PAD, EMB)
    deg_t = deg.reshape(NC, N_PAD).T
    return _tc_finish(x, acc, deg_t, W, b.reshape(1, EMB))


# speedup vs baseline: 10.0746x; 1.5644x over previous
import jax
import jax.numpy as jnp
from jax import lax
from jax.experimental import pallas as pl
from jax.experimental.pallas import tpu as pltpu
from jax.experimental.pallas import tpu_sc as plsc

N_NODES = 10000
N_EDGES = 320000
EMB = 128

NC = 2
NS = 16
NW = NC * NS
EDGES_PER_TILE = N_EDGES // NW
CHUNK = 80
STEPS = EDGES_PER_TILE // CHUNK
N_PAD = 10240
ROWS_PER_TILE = N_PAD // NS
HROWS = N_PAD // EMB
VW = 16


def _fill_rows(ref, nrows, ncols, value):
    v = jnp.full((VW,), value, jnp.float32)

    def body(r, carry):
        for j in range(ncols // VW):
            ref[r, pl.ds(j * VW, VW)] = v
        return carry

    lax.fori_loop(0, nrows, body, 0)


def _sc_body(x_hbm, src_hbm, dst_hbm, acc_out, deg_out,
             src_a, dst_a, src_b, dst_b, rows_a, rows_b, hist_v, iota_v,
             sem_a, sem_b, acc_sp, deg_sp):
    c = lax.axis_index("c")
    s = lax.axis_index("s")
    wid = c * NS + s

    _fill_rows(hist_v, HROWS, EMB, 0.0)
    row0 = s * ROWS_PER_TILE
    for k in range(ROWS_PER_TILE // HROWS):
        pltpu.sync_copy(hist_v, acc_sp.at[pl.ds(row0 + k * HROWS, HROWS), :])

    @pl.when(s < HROWS // 8)
    def _zero_deg():
        pltpu.sync_copy(hist_v.at[pl.ds(0, 8), :],
                        deg_sp.at[pl.ds(s * 8, 8), :])

    for j in range(HROWS // VW):
        iota_v[pl.ds(j * VW, VW)] = lax.iota(jnp.int32, VW) + j * VW
    plsc.subcore_barrier()

    ebase = wid * EDGES_PER_TILE
    ones = jnp.full((VW,), 1.0, jnp.float32)

    def load_idx(i, sv, dv):
        off = ebase + i * CHUNK
        pltpu.sync_copy(src_hbm.at[pl.ds(off, CHUNK)], sv)
        pltpu.sync_copy(dst_hbm.at[pl.ds(off, CHUNK)], dv)

    def consume(dv, rv):
        for j in range(CHUNK // VW):
            d = dv[pl.ds(j * VW, VW)]
            plsc.addupdate_scatter(
                hist_v,
                (lax.shift_right_logical(d, 7), lax.bitwise_and(d, 127)),
                ones)
        pltpu.sync_copy(rv, acc_sp.at[dv], add=True)

    load_idx(0, src_a, dst_a)
    pltpu.async_copy(x_hbm.at[src_a], rows_a, sem_a)

    def pair(k, carry):
        i = 2 * k
        load_idx(i + 1, src_b, dst_b)
        gb = pltpu.async_copy(x_hbm.at[src_b], rows_b, sem_b)
        pltpu.make_async_copy(x_hbm.at[src_a], rows_a, sem_a).wait()
        consume(dst_a, rows_a)
        load_idx(i + 2, src_a, dst_a)
        pltpu.async_copy(x_hbm.at[src_a], rows_a, sem_a)
        gb.wait()
        consume(dst_b, rows_b)
        return carry

    lax.fori_loop(0, (STEPS - 1) // 2, pair, 0)
    pltpu.make_async_copy(x_hbm.at[src_a], rows_a, sem_a).wait()
    consume(dst_a, rows_a)

    pltpu.sync_copy(hist_v, deg_sp.at[iota_v], add=True)
    plsc.subcore_barrier()

    out0 = c * N_PAD + row0
    pltpu.sync_copy(acc_sp.at[pl.ds(row0, ROWS_PER_TILE), :],
                    acc_out.at[pl.ds(out0, ROWS_PER_TILE), :])

    @pl.when(s < HROWS // 8)
    def _copy_deg():
        pltpu.sync_copy(deg_sp.at[pl.ds(s * 8, 8), :],
                        deg_out.at[pl.ds(c * HROWS + s * 8, 8), :])


def _sc_aggregate(x, src, dst):
    mesh = plsc.VectorSubcoreMesh(core_axis_name="c", subcore_axis_name="s")
    f = pl.kernel(
        _sc_body,
        out_type=(
            jax.ShapeDtypeStruct((NC * N_PAD, EMB), jnp.float32),
            jax.ShapeDtypeStruct((NC * HROWS, EMB), jnp.float32),
        ),
        mesh=mesh,
        compiler_params=pltpu.CompilerParams(needs_layout_passes=False),
        scratch_types=[
            pltpu.VMEM((CHUNK,), jnp.int32),
            pltpu.VMEM((CHUNK,), jnp.int32),
            pltpu.VMEM((CHUNK,), jnp.int32),
            pltpu.VMEM((CHUNK,), jnp.int32),
            pltpu.VMEM((CHUNK, EMB), jnp.float32),
            pltpu.VMEM((CHUNK, EMB), jnp.float32),
            pltpu.VMEM((HROWS, EMB), jnp.float32),
            pltpu.VMEM((HROWS,), jnp.int32),
            pltpu.SemaphoreType.DMA,
            pltpu.SemaphoreType.DMA,
            pltpu.VMEM_SHARED((N_PAD, EMB), jnp.float32),
            pltpu.VMEM_SHARED((HROWS, EMB), jnp.float32),
        ],
    )
    return f(x, src, dst)


def _tc_body(x_ref, acc_ref, deg_ref, w_ref, b_ref, out_ref):
    neigh = acc_ref[0] + acc_ref[1]
    deg = deg_ref[:, 0:1] + deg_ref[:, 1:2]
    agg = (x_ref[...] + neigh) * lax.rsqrt(deg + 1.0)
    h = lax.dot_general(agg, w_ref[...], (((1,), (1,)), ((), ())),
                        preferred_element_type=jnp.float32) + b_ref[...]
    out_ref[...] = jnp.where(h > 0, h, 0.2 * h)


def _tc_finish(x, acc, deg_t, W, b):
    R = 2000
    grid = (N_NODES // R,)
    return pl.pallas_call(
        _tc_body,
        grid=grid,
        in_specs=[
            pl.BlockSpec((R, EMB), lambda i: (i, 0)),
            pl.BlockSpec((NC, R, EMB), lambda i: (0, i, 0)),
            pl.BlockSpec((R, NC), lambda i: (i, 0)),
            pl.BlockSpec((EMB, EMB), lambda i: (0, 0)),
            pl.BlockSpec((1, EMB), lambda i: (0, 0)),
        ],
        out_specs=pl.BlockSpec((R, EMB), lambda i: (i, 0)),
        out_shape=jax.ShapeDtypeStruct((N_NODES, EMB), jnp.float32),
    )(x, acc, deg_t, W, b)


@jax.jit
def kernel(x, edge_index, W, b):
    src = edge_index[0]
    dst = edge_index[1]
    acc, deg = _sc_aggregate(x, src, dst)
    acc = acc.reshape(NC, N_PAD, EMB)
    deg_t = deg.reshape(NC, N_PAD).T
    return _tc_finish(x, acc, deg_t, W, b.reshape(1, EMB))


# speedup vs baseline: 13.7441x; 1.3642x over previous
import jax
import jax.numpy as jnp
from jax import lax
from jax.experimental import pallas as pl
from jax.experimental.pallas import tpu as pltpu
from jax.experimental.pallas import tpu_sc as plsc

N_NODES = 10000
N_EDGES = 320000
EMB = 128

NC = 2
NS = 16
NW = NC * NS
EDGES_PER_TILE = N_EDGES // NW
CHUNK = 80
STEPS = EDGES_PER_TILE // CHUNK
GCHUNKS = 25
GEDGES = GCHUNKS * CHUNK
NGROUPS = STEPS // GCHUNKS
N_PAD = 10240
ROWS_PER_TILE = N_PAD // NS
HROWS = N_PAD // EMB
VW = 16


def _fill_rows(ref, nrows, ncols, value):
    v = jnp.full((VW,), value, jnp.float32)

    def body(r, carry):
        for j in range(ncols // VW):
            ref[r, pl.ds(j * VW, VW)] = v
        return carry

    lax.fori_loop(0, nrows, body, 0)


def _sc_body(x_hbm, src_hbm, dst_hbm, acc_out, deg_out,
             src_g0, dst_g0, src_g1, dst_g1, rows_a, rows_b, hist_v, iota_v,
             sem_a, sem_b, isem_s0, isem_d0, isem_s1, isem_d1,
             acc_sp, deg_sp):
    c = lax.axis_index("c")
    s = lax.axis_index("s")
    wid = c * NS + s

    _fill_rows(hist_v, HROWS, EMB, 0.0)
    row0 = s * ROWS_PER_TILE
    for k in range(ROWS_PER_TILE // HROWS):
        pltpu.sync_copy(hist_v, acc_sp.at[pl.ds(row0 + k * HROWS, HROWS), :])

    @pl.when(s < HROWS // 8)
    def _zero_deg():
        pltpu.sync_copy(hist_v.at[pl.ds(0, 8), :],
                        deg_sp.at[pl.ds(s * 8, 8), :])

    for j in range(HROWS // VW):
        iota_v[pl.ds(j * VW, VW)] = lax.iota(jnp.int32, VW) + j * VW
    plsc.subcore_barrier()

    ebase = wid * EDGES_PER_TILE
    ones = jnp.full((VW,), 1.0, jnp.float32)

    def consume(dv, rv):
        for j in range(CHUNK // VW):
            d = dv[pl.ds(j * VW, VW)]
            plsc.addupdate_scatter(
                hist_v,
                (lax.shift_right_logical(d, 7), lax.bitwise_and(d, 127)),
                ones)
        pltpu.sync_copy(rv, acc_sp.at[dv], add=True)

    src_g = (src_g0, src_g1)
    dst_g = (dst_g0, dst_g1)
    isems = ((isem_s0, isem_d0), (isem_s1, isem_d1))

    def prefetch(g):
        b = g % 2
        off = ebase + g * GEDGES
        pltpu.async_copy(src_hbm.at[pl.ds(off, GEDGES)], src_g[b], isems[b][0])
        pltpu.async_copy(dst_hbm.at[pl.ds(off, GEDGES)], dst_g[b], isems[b][1])

    prefetch(0)
    for g in range(NGROUPS):
        b = g % 2
        sg, dg = src_g[b], dst_g[b]
        pltpu.make_async_copy(src_hbm.at[pl.ds(0, GEDGES)], sg,
                              isems[b][0]).wait()
        pltpu.make_async_copy(dst_hbm.at[pl.ds(0, GEDGES)], dg,
                              isems[b][1]).wait()
        if g + 1 < NGROUPS:
            prefetch(g + 1)

        def sv(i):
            return sg.at[pl.ds(i * CHUNK, CHUNK)]

        def dv(i):
            return dg.at[pl.ds(i * CHUNK, CHUNK)]

        pltpu.async_copy(x_hbm.at[sv(0)], rows_a, sem_a)

        def pair(k, carry):
            i = 2 * k
            gb = pltpu.async_copy(x_hbm.at[sv(i + 1)], rows_b, sem_b)
            pltpu.make_async_copy(x_hbm.at[sv(i)], rows_a, sem_a).wait()
            consume(dv(i), rows_a)
            pltpu.async_copy(x_hbm.at[sv(i + 2)], rows_a, sem_a)
            gb.wait()
            consume(dv(i + 1), rows_b)
            return carry

        lax.fori_loop(0, (GCHUNKS - 1) // 2, pair, 0)
        pltpu.make_async_copy(x_hbm.at[sv(GCHUNKS - 1)], rows_a, sem_a).wait()
        consume(dv(GCHUNKS - 1), rows_a)

    pltpu.sync_copy(hist_v, deg_sp.at[iota_v], add=True)
    plsc.subcore_barrier()

    out0 = c * N_PAD + row0
    pltpu.sync_copy(acc_sp.at[pl.ds(row0, ROWS_PER_TILE), :],
                    acc_out.at[pl.ds(out0, ROWS_PER_TILE), :])

    @pl.when(s < HROWS // 8)
    def _copy_deg():
        pltpu.sync_copy(deg_sp.at[pl.ds(s * 8, 8), :],
                        deg_out.at[pl.ds(c * HROWS + s * 8, 8), :])


def _sc_aggregate(x, src, dst):
    mesh = plsc.VectorSubcoreMesh(core_axis_name="c", subcore_axis_name="s")
    f = pl.kernel(
        _sc_body,
        out_type=(
            jax.ShapeDtypeStruct((NC * N_PAD, EMB), jnp.float32),
            jax.ShapeDtypeStruct((NC * HROWS, EMB), jnp.float32),
        ),
        mesh=mesh,
        compiler_params=pltpu.CompilerParams(needs_layout_passes=False),
        scratch_types=[
            pltpu.VMEM((GEDGES,), jnp.int32),
            pltpu.VMEM((GEDGES,), jnp.int32),
            pltpu.VMEM((GEDGES,), jnp.int32),
            pltpu.VMEM((GEDGES,), jnp.int32),
            pltpu.VMEM((CHUNK, EMB), jnp.float32),
            pltpu.VMEM((CHUNK, EMB), jnp.float32),
            pltpu.VMEM((HROWS, EMB), jnp.float32),
            pltpu.VMEM((HROWS,), jnp.int32),
            pltpu.SemaphoreType.DMA,
            pltpu.SemaphoreType.DMA,
            pltpu.SemaphoreType.DMA,
            pltpu.SemaphoreType.DMA,
            pltpu.SemaphoreType.DMA,
            pltpu.SemaphoreType.DMA,
            pltpu.VMEM_SHARED((N_PAD, EMB), jnp.float32),
            pltpu.VMEM_SHARED((HROWS, EMB), jnp.float32),
        ],
    )
    return f(x, src, dst)


def _tc_body(x_ref, acc_ref, deg_ref, w_ref, b_ref, out_ref):
    neigh = acc_ref[0] + acc_ref[1]
    deg = deg_ref[:, 0:1] + deg_ref[:, 1:2]
    agg = (x_ref[...] + neigh) * lax.rsqrt(deg + 1.0)
    h = lax.dot_general(agg, w_ref[...], (((1,), (1,)), ((), ())),
                        preferred_element_type=jnp.float32) + b_ref[...]
    out_ref[...] = jnp.where(h > 0, h, 0.2 * h)


def _tc_finish(x, acc, deg_t, W, b):
    R = 2000
    grid = (N_NODES // R,)
    return pl.pallas_call(
        _tc_body,
        grid=grid,
        in_specs=[
            pl.BlockSpec((R, EMB), lambda i: (i, 0)),
            pl.BlockSpec((NC, R, EMB), lambda i: (0, i, 0)),
            pl.BlockSpec((R, NC), lambda i: (i, 0)),
            pl.BlockSpec((EMB, EMB), lambda i: (0, 0)),
            pl.BlockSpec((1, EMB), lambda i: (0, 0)),
        ],
        out_specs=pl.BlockSpec((R, EMB), lambda i: (i, 0)),
        out_shape=jax.ShapeDtypeStruct((N_NODES, EMB), jnp.float32),
    )(x, acc, deg_t, W, b)


@jax.jit
def kernel(x, edge_index, W, b):
    src = edge_index[0]
    dst = edge_index[1]
    acc, deg = _sc_aggregate(x, src, dst)
    acc = acc.reshape(NC, N_PAD, EMB)
    deg_t = deg.reshape(NC, N_PAD).T
    return _tc_finish(x, acc, deg_t, W, b.reshape(1, EMB))
